# baseline (speedup 1.0000x reference)
import functools

import jax
import jax.numpy as jnp
from jax.experimental import pallas as pl
from jax.experimental.pallas import tpu as pltpu

_EMB = 512
_M = 1024
_UNROLL = 16


def _gather_body(idx_ref, w_ref, out_ref, sem, *, nsteps):
    k = pl.program_id(0)
    base = k * _M

    def issue(u, carry):
        m0 = base + u * _UNROLL
        for j in range(_UNROLL):
            row = idx_ref[m0 + j]
            pltpu.make_async_copy(
                w_ref.at[pl.ds(row, 1)],
                out_ref.at[pl.ds(m0 + j, 1)],
                sem,
            ).start()
        return carry

    jax.lax.fori_loop(0, _M // _UNROLL, issue, 0)

    @pl.when(k > 0)
    def _wait_prev():
        pltpu.make_async_copy(
            w_ref.at[pl.ds(0, _M)], out_ref.at[pl.ds(0, _M)], sem
        ).wait()

    @pl.when(k == nsteps - 1)
    def _wait_last():
        pltpu.make_async_copy(
            w_ref.at[pl.ds(0, _M)], out_ref.at[pl.ds(0, _M)], sem
        ).wait()


def kernel(input, weight):
    bptt, bsize = input.shape
    n = bptt * bsize
    idx = input.reshape(n).astype(jnp.int32)
    nsteps = n // _M

    grid_spec = pltpu.PrefetchScalarGridSpec(
        num_scalar_prefetch=1,
        grid=(nsteps,),
        in_specs=[pl.BlockSpec(memory_space=pl.ANY)],
        out_specs=pl.BlockSpec(memory_space=pl.ANY),
        scratch_shapes=[pltpu.SemaphoreType.DMA],
    )
    out = pl.pallas_call(
        functools.partial(_gather_body, nsteps=nsteps),
        grid_spec=grid_spec,
        out_shape=jax.ShapeDtypeStruct((n, _EMB), jnp.float32),
        compiler_params=pltpu.CompilerParams(
            dimension_semantics=("arbitrary",),
            disable_bounds_checks=True,
        ),
    )(idx, weight)
    return out.reshape(bptt, bsize, _EMB)


# speedup vs baseline: 8.8272x; 8.8272x over previous
import functools

import jax
import jax.numpy as jnp
from jax.experimental import pallas as pl
from jax.experimental.pallas import tpu as pltpu

_EMB = 512
_M = 1024
_UNROLL = 16


def _gather_body(idx_ref, w_ref, out_ref, buf0, buf1, gsem, wsem, *, nsteps):
    k = pl.program_id(0)
    bufs = (buf0, buf1)

    for p in (0, 1):
        buf = bufs[p]

        @pl.when(jnp.logical_and(k < nsteps, k % 2 == p))
        def _issue():
            @pl.when(k >= 2)
            def _wait_write():
                pltpu.make_async_copy(buf, out_ref.at[pl.ds(0, _M)], wsem.at[p]).wait()

            base = k * _M

            def issue(u, carry):
                m0 = u * _UNROLL
                for j in range(_UNROLL):
                    row = idx_ref[base + m0 + j]
                    pltpu.make_async_copy(
                        w_ref.at[pl.ds(row, 1)],
                        buf.at[pl.ds(m0 + j, 1)],
                        gsem.at[p],
                    ).start()
                return carry

            jax.lax.fori_loop(0, _M // _UNROLL, issue, 0)

        @pl.when(jnp.logical_and(k >= 1, k % 2 == p))
        def _flush_prev():
            prev = bufs[1 - p]
            pltpu.make_async_copy(
                w_ref.at[pl.ds(0, _M)], prev, gsem.at[1 - p]
            ).wait()
            pltpu.make_async_copy(
                prev, out_ref.at[pl.ds((k - 1) * _M, _M)], wsem.at[1 - p]
            ).start()

    @pl.when(k == nsteps)
    def _final():
        pltpu.make_async_copy(buf0, out_ref.at[pl.ds(0, _M)], wsem.at[0]).wait()
        pltpu.make_async_copy(buf1, out_ref.at[pl.ds(0, _M)], wsem.at[1]).wait()


def kernel(input, weight):
    bptt, bsize = input.shape
    n = bptt * bsize
    idx = input.reshape(n).astype(jnp.int32)
    nsteps = n // _M

    grid_spec = pltpu.PrefetchScalarGridSpec(
        num_scalar_prefetch=1,
        grid=(nsteps + 1,),
        in_specs=[pl.BlockSpec(memory_space=pl.ANY)],
        out_specs=pl.BlockSpec(memory_space=pl.ANY),
        scratch_shapes=[
            pltpu.VMEM((_M, _EMB), jnp.float32),
            pltpu.VMEM((_M, _EMB), jnp.float32),
            pltpu.SemaphoreType.DMA((2,)),
            pltpu.SemaphoreType.DMA((2,)),
        ],
    )
    out = pl.pallas_call(
        functools.partial(_gather_body, nsteps=nsteps),
        grid_spec=grid_spec,
        out_shape=jax.ShapeDtypeStruct((n, _EMB), jnp.float32),
        compiler_params=pltpu.CompilerParams(
            dimension_semantics=("arbitrary",),
            disable_bounds_checks=True,
        ),
    )(idx, weight)
    return out.reshape(bptt, bsize, _EMB)


# speedup vs baseline: 10.6152x; 1.2026x over previous
import functools

import jax
import jax.numpy as jnp
from jax.experimental import pallas as pl
from jax.experimental.pallas import tpu as pltpu

_EMB = 512
_M = 512


def _gather_body(idx_ref, w_ref, out_ref, sem, *, nsteps):
    k = pl.program_id(0)
    base = k * _M

    for m in range(_M):
        row = idx_ref[base + m]
        pltpu.make_async_copy(
            w_ref.at[pl.ds(row, 1)],
            out_ref.at[pl.ds(m, 1)],
            sem,
        ).start()

    pltpu.make_async_copy(
        w_ref.at[pl.ds(0, _M)], out_ref.at[pl.ds(0, _M)], sem
    ).wait()


def kernel(input, weight):
    bptt, bsize = input.shape
    n = bptt * bsize
    idx = input.reshape(n).astype(jnp.int32)
    nsteps = n // _M

    grid_spec = pltpu.PrefetchScalarGridSpec(
        num_scalar_prefetch=1,
        grid=(nsteps,),
        in_specs=[pl.BlockSpec(memory_space=pl.ANY)],
        out_specs=pl.BlockSpec(
            (_M, _EMB),
            lambda k, idx_ref: (k, 0),
        ),
        scratch_shapes=[pltpu.SemaphoreType.DMA],
    )
    out = pl.pallas_call(
        functools.partial(_gather_body, nsteps=nsteps),
        grid_spec=grid_spec,
        out_shape=jax.ShapeDtypeStruct((n, _EMB), jnp.float32),
        compiler_params=pltpu.CompilerParams(
            dimension_semantics=("arbitrary",),
            disable_bounds_checks=True,
        ),
    )(idx, weight)
    return out.reshape(bptt, bsize, _EMB)


# speedup vs baseline: 11.3348x; 1.0678x over previous
import functools

import jax
import jax.numpy as jnp
from jax.experimental import pallas as pl
from jax.experimental.pallas import tpu as pltpu

_EMB = 512
_M = 512


def _gather_body(idx_ref, w_ref, out_ref, buf0, buf1, gsem, wsem, *, nsteps):
    k = pl.program_id(0)
    bufs = (buf0, buf1)

    for p in (0, 1):
        buf = bufs[p]

        @pl.when(jnp.logical_and(k < nsteps, k % 2 == p))
        def _issue():
            @pl.when(k >= 2)
            def _wait_write():
                pltpu.make_async_copy(buf, out_ref.at[pl.ds(0, _M)], wsem.at[p]).wait()

            base = k * _M
            for m in range(_M):
                row = idx_ref[base + m]
                pltpu.make_async_copy(
                    w_ref.at[pl.ds(row, 1)],
                    buf.at[pl.ds(m, 1)],
                    gsem.at[p],
                ).start()

        @pl.when(jnp.logical_and(k >= 1, k % 2 == p))
        def _flush_prev():
            prev = bufs[1 - p]
            pltpu.make_async_copy(
                w_ref.at[pl.ds(0, _M)], prev, gsem.at[1 - p]
            ).wait()
            pltpu.make_async_copy(
                prev, out_ref.at[pl.ds((k - 1) * _M, _M)], wsem.at[1 - p]
            ).start()

    @pl.when(k == nsteps)
    def _final():
        pltpu.make_async_copy(buf0, out_ref.at[pl.ds(0, _M)], wsem.at[0]).wait()
        pltpu.make_async_copy(buf1, out_ref.at[pl.ds(0, _M)], wsem.at[1]).wait()


def kernel(input, weight):
    bptt, bsize = input.shape
    n = bptt * bsize
    idx = input.reshape(n).astype(jnp.int32)
    nsteps = n // _M

    grid_spec = pltpu.PrefetchScalarGridSpec(
        num_scalar_prefetch=1,
        grid=(nsteps + 1,),
        in_specs=[pl.BlockSpec(memory_space=pl.ANY)],
        out_specs=pl.BlockSpec(memory_space=pl.ANY),
        scratch_shapes=[
            pltpu.VMEM((_M, _EMB), jnp.float32),
            pltpu.VMEM((_M, _EMB), jnp.float32),
            pltpu.SemaphoreType.DMA((2,)),
            pltpu.SemaphoreType.DMA((2,)),
        ],
    )
    out = pl.pallas_call(
        functools.partial(_gather_body, nsteps=nsteps),
        grid_spec=grid_spec,
        out_shape=jax.ShapeDtypeStruct((n, _EMB), jnp.float32),
        compiler_params=pltpu.CompilerParams(
            dimension_semantics=("arbitrary",),
            disable_bounds_checks=True,
        ),
    )(idx, weight)
    return out.reshape(bptt, bsize, _EMB)


# speedup vs baseline: 11.3370x; 1.0002x over previous
import functools

import jax
import jax.numpy as jnp
from jax.experimental import pallas as pl
from jax.experimental.pallas import tpu as pltpu

_EMB = 512
_M = 512


def _gather_body(idx_ref, w_ref, out_ref, buf0, buf1, gsem, wsem, *, nsteps):
    k = pl.program_id(0)
    bufs = (buf0, buf1)

    for p in (0, 1):
        buf = bufs[p]

        @pl.when(jnp.logical_and(k < nsteps, k % 2 == p))
        def _issue():
            @pl.when(k >= 2)
            def _wait_write():
                pltpu.make_async_copy(buf, out_ref.at[pl.ds(0, _M)], wsem.at[p]).wait()

            base = k * _M
            for m in range(_M):
                row = idx_ref[base + m]
                pltpu.make_async_copy(
                    w_ref.at[pl.ds(row, 1)],
                    buf.at[pl.ds(m, 1)],
                    gsem.at[2 * p + (m % 2)],
                ).start()

        @pl.when(jnp.logical_and(k >= 1, k % 2 == p))
        def _flush_prev():
            prev = bufs[1 - p]
            half = w_ref.at[pl.ds(0, _M // 2)], prev.at[pl.ds(0, _M // 2)]
            pltpu.make_async_copy(*half, gsem.at[2 * (1 - p)]).wait()
            pltpu.make_async_copy(*half, gsem.at[2 * (1 - p) + 1]).wait()
            pltpu.make_async_copy(
                prev, out_ref.at[pl.ds((k - 1) * _M, _M)], wsem.at[1 - p]
            ).start()

    @pl.when(k == nsteps)
    def _final():
        pltpu.make_async_copy(buf0, out_ref.at[pl.ds(0, _M)], wsem.at[0]).wait()
        pltpu.make_async_copy(buf1, out_ref.at[pl.ds(0, _M)], wsem.at[1]).wait()


def kernel(input, weight):
    bptt, bsize = input.shape
    n = bptt * bsize
    idx = input.reshape(n).astype(jnp.int32)
    nsteps = n // _M

    grid_spec = pltpu.PrefetchScalarGridSpec(
        num_scalar_prefetch=1,
        grid=(nsteps + 1,),
        in_specs=[pl.BlockSpec(memory_space=pl.ANY)],
        out_specs=pl.BlockSpec(memory_space=pl.ANY),
        scratch_shapes=[
            pltpu.VMEM((_M, _EMB), jnp.float32),
            pltpu.VMEM((_M, _EMB), jnp.float32),
            pltpu.SemaphoreType.DMA((4,)),
            pltpu.SemaphoreType.DMA((2,)),
        ],
    )
    out = pl.pallas_call(
        functools.partial(_gather_body, nsteps=nsteps),
        grid_spec=grid_spec,
        out_shape=jax.ShapeDtypeStruct((n, _EMB), jnp.float32),
        compiler_params=pltpu.CompilerParams(
            dimension_semantics=("arbitrary",),
            disable_bounds_checks=True,
        ),
    )(idx, weight)
    return out.reshape(bptt, bsize, _EMB)


# speedup vs baseline: 12.6376x; 1.1147x over previous
import functools

import jax
import jax.numpy as jnp
from jax.experimental import pallas as pl
from jax.experimental.pallas import tpu as pltpu

_EMB = 512
_M = 1024


def _gather_body(idx_ref, w_ref, out_ref, buf0, buf1, gsem, wsem, *, nsteps):
    k = pl.program_id(0)
    bufs = (buf0, buf1)

    for p in (0, 1):
        buf = bufs[p]

        @pl.when(jnp.logical_and(k < nsteps, k % 2 == p))
        def _issue():
            @pl.when(k >= 2)
            def _wait_write():
                pltpu.make_async_copy(buf, out_ref.at[pl.ds(0, _M)], wsem.at[p]).wait()

            base = k * _M
            for m in range(_M):
                row = idx_ref[base + m]
                pltpu.make_async_copy(
                    w_ref.at[pl.ds(row, 1)],
                    buf.at[pl.ds(m, 1)],
                    gsem.at[2 * p + (m % 2)],
                ).start()

        @pl.when(jnp.logical_and(k >= 1, k % 2 == p))
        def _flush_prev():
            prev = bufs[1 - p]
            half = w_ref.at[pl.ds(0, _M // 2)], prev.at[pl.ds(0, _M // 2)]
            pltpu.make_async_copy(*half, gsem.at[2 * (1 - p)]).wait()
            pltpu.make_async_copy(*half, gsem.at[2 * (1 - p) + 1]).wait()
            pltpu.make_async_copy(
                prev, out_ref.at[pl.ds((k - 1) * _M, _M)], wsem.at[1 - p]
            ).start()

    @pl.when(k == nsteps)
    def _final():
        pltpu.make_async_copy(buf0, out_ref.at[pl.ds(0, _M)], wsem.at[0]).wait()
        pltpu.make_async_copy(buf1, out_ref.at[pl.ds(0, _M)], wsem.at[1]).wait()


def kernel(input, weight):
    bptt, bsize = input.shape
    n = bptt * bsize
    idx = input.reshape(n).astype(jnp.int32)
    nsteps = n // _M

    grid_spec = pltpu.PrefetchScalarGridSpec(
        num_scalar_prefetch=1,
        grid=(nsteps + 1,),
        in_specs=[pl.BlockSpec(memory_space=pl.ANY)],
        out_specs=pl.BlockSpec(memory_space=pl.ANY),
        scratch_shapes=[
            pltpu.VMEM((_M, _EMB), jnp.float32),
            pltpu.VMEM((_M, _EMB), jnp.float32),
            pltpu.SemaphoreType.DMA((4,)),
            pltpu.SemaphoreType.DMA((2,)),
        ],
    )
    out = pl.pallas_call(
        functools.partial(_gather_body, nsteps=nsteps),
        grid_spec=grid_spec,
        out_shape=jax.ShapeDtypeStruct((n, _EMB), jnp.float32),
        compiler_params=pltpu.CompilerParams(
            dimension_semantics=("arbitrary",),
            disable_bounds_checks=True,
        ),
    )(idx, weight)
    return out.reshape(bptt, bsize, _EMB)


# speedup vs baseline: 13.3506x; 1.0564x over previous
import functools

import jax
import jax.numpy as jnp
from jax.experimental import pallas as pl
from jax.experimental.pallas import tpu as pltpu

_EMB = 512
_M = 2048


def _gather_body(idx_ref, w_ref, out_ref, buf0, buf1, gsem, wsem, *, nsteps):
    k = pl.program_id(0)
    bufs = (buf0, buf1)

    for p in (0, 1):
        buf = bufs[p]

        @pl.when(jnp.logical_and(k < nsteps, k % 2 == p))
        def _issue():
            @pl.when(k >= 2)
            def _wait_write():
                pltpu.make_async_copy(buf, out_ref.at[pl.ds(0, _M)], wsem.at[p]).wait()

            base = k * _M
            for m in range(_M):
                row = idx_ref[base + m]
                pltpu.make_async_copy(
                    w_ref.at[pl.ds(row, 1)],
                    buf.at[pl.ds(m, 1)],
                    gsem.at[2 * p + (m % 2)],
                ).start()

        @pl.when(jnp.logical_and(k >= 1, k % 2 == p))
        def _flush_prev():
            prev = bufs[1 - p]
            half = w_ref.at[pl.ds(0, _M // 2)], prev.at[pl.ds(0, _M // 2)]
            pltpu.make_async_copy(*half, gsem.at[2 * (1 - p)]).wait()
            pltpu.make_async_copy(*half, gsem.at[2 * (1 - p) + 1]).wait()
            pltpu.make_async_copy(
                prev, out_ref.at[pl.ds((k - 1) * _M, _M)], wsem.at[1 - p]
            ).start()

    @pl.when(k == nsteps)
    def _final():
        pltpu.make_async_copy(buf0, out_ref.at[pl.ds(0, _M)], wsem.at[0]).wait()
        pltpu.make_async_copy(buf1, out_ref.at[pl.ds(0, _M)], wsem.at[1]).wait()


def kernel(input, weight):
    bptt, bsize = input.shape
    n = bptt * bsize
    idx = input.reshape(n).astype(jnp.int32)
    nsteps = n // _M

    grid_spec = pltpu.PrefetchScalarGridSpec(
        num_scalar_prefetch=1,
        grid=(nsteps + 1,),
        in_specs=[pl.BlockSpec(memory_space=pl.ANY)],
        out_specs=pl.BlockSpec(memory_space=pl.ANY),
        scratch_shapes=[
            pltpu.VMEM((_M, _EMB), jnp.float32),
            pltpu.VMEM((_M, _EMB), jnp.float32),
            pltpu.SemaphoreType.DMA((4,)),
            pltpu.SemaphoreType.DMA((2,)),
        ],
    )
    out = pl.pallas_call(
        functools.partial(_gather_body, nsteps=nsteps),
        grid_spec=grid_spec,
        out_shape=jax.ShapeDtypeStruct((n, _EMB), jnp.float32),
        compiler_params=pltpu.CompilerParams(
            dimension_semantics=("arbitrary",),
            disable_bounds_checks=True,
        ),
    )(idx, weight)
    return out.reshape(bptt, bsize, _EMB)
